# baseline (speedup 1.0000x reference)
import functools

import jax
import jax.numpy as jnp
from jax.experimental import pallas as pl
import jax.experimental.pallas.tpu as pltpu

Q = 3136
K = 8192
D = 1536
QB = 448
NQB = Q // QB
KB = 2048
NKB = K // KB
NN = 9
CW = 32
OUT_W = 16


def _prep_q_body(x_ref, xs_ref, x2_ref):
    x = x_ref[...]
    n = jnp.sqrt(jnp.sum(x * x, axis=1, keepdims=True))
    xn = x / jnp.maximum(n, 1e-12)
    x2_ref[...] = jnp.sum(xn * xn, axis=1, keepdims=True)
    xs_ref[...] = -2.0 * xn


def _prep_k_body(x_ref, xn_ref, x2_ref):
    x = x_ref[...]
    n = jnp.sqrt(jnp.sum(x * x, axis=1, keepdims=True))
    xn = x / jnp.maximum(n, 1e-12)
    xn_ref[...] = xn
    x2_ref[...] = jnp.sum(xn * xn, axis=1)[None, :]


def _knn_body(
    qs_ref, q2_ref, mn_ref, m2_ref, kd_ref, ki_ref, d2a_ref, d2b_ref, cv_ref, ci_ref
):
    jj = pl.program_id(1)

    def compute_into(dst_ref):
        dot2 = jax.lax.dot_general(
            qs_ref[...], mn_ref[...], (((1,), (1,)), ((), ())),
            preferred_element_type=jnp.float32,
        )
        dst_ref[...] = jnp.maximum((q2_ref[...] + m2_ref[...]) + dot2, 0.0)

    def extract_from(src_ref):
        j = jnp.maximum(jj - 1, 0)
        cur = src_ref[...]
        iota = jax.lax.broadcasted_iota(jnp.int32, (QB, KB), 1).astype(jnp.float32)
        base = (j * KB).astype(jnp.float32)
        vals, idxs = [], []
        for _ in range(NN):
            mv = jnp.min(cur, axis=1, keepdims=True)
            e = jnp.where(cur == mv, iota, jnp.float32(KB))
            ii = jnp.min(e, axis=1, keepdims=True)
            vals.append(mv)
            idxs.append(ii + base)
            cur = jnp.where(e == ii, jnp.float32(jnp.inf), cur)
        pad = CW - NN
        cv = jnp.concatenate(
            vals + [jnp.full((QB, pad), jnp.inf, jnp.float32)], 1
        )
        ci = jnp.concatenate(idxs + [jnp.zeros((QB, pad), jnp.float32)], 1)
        cv_ref[j] = cv
        ci_ref[j] = ci

    @pl.when(jj % 2 == 0)
    def _():
        compute_into(d2a_ref)
        extract_from(d2b_ref)

    @pl.when(jj % 2 == 1)
    def _():
        compute_into(d2b_ref)
        extract_from(d2a_ref)

    @pl.when(jj == NKB)
    def _():
        v = jnp.concatenate([cv_ref[c] for c in range(NKB)], axis=1)
        gi = jnp.concatenate([ci_ref[c] for c in range(NKB)], axis=1)
        pos_id = jax.lax.broadcasted_iota(jnp.int32, (QB, NKB * CW), 1).astype(
            jnp.float32
        )
        out_v, out_i = [], []
        for _ in range(NN):
            mv = jnp.min(v, axis=1, keepdims=True)
            e = jnp.where(v == mv, pos_id, jnp.float32(NKB * CW))
            pos = jnp.min(e, axis=1, keepdims=True)
            hit = e == pos
            ii = jnp.min(jnp.where(hit, gi, jnp.float32(K)), axis=1, keepdims=True)
            out_v.append(mv)
            out_i.append(ii)
            v = jnp.where(hit, jnp.float32(jnp.inf), v)
        opad = OUT_W - NN
        dv = jnp.concatenate(out_v + [jnp.zeros((QB, opad), jnp.float32)], 1)
        di = jnp.concatenate(out_i + [jnp.zeros((QB, opad), jnp.float32)], 1)
        kd_ref[...] = jnp.sqrt(dv + 1e-12)
        ki_ref[...] = di.astype(jnp.int32)


@functools.partial(jax.jit, static_argnames=("interpret",))
def _knn(queries, keys, interpret=False):
    qs, q2 = pl.pallas_call(
        _prep_q_body,
        grid=(NQB,),
        in_specs=[pl.BlockSpec((QB, D), lambda i: (i, 0))],
        out_specs=[
            pl.BlockSpec((QB, D), lambda i: (i, 0)),
            pl.BlockSpec((QB, 1), lambda i: (i, 0)),
        ],
        out_shape=[
            jax.ShapeDtypeStruct((Q, D), jnp.float32),
            jax.ShapeDtypeStruct((Q, 1), jnp.float32),
        ],
        interpret=interpret,
    )(queries)
    mn, m2 = pl.pallas_call(
        _prep_k_body,
        grid=(NKB,),
        in_specs=[pl.BlockSpec((KB, D), lambda j: (j, 0))],
        out_specs=[
            pl.BlockSpec((KB, D), lambda j: (j, 0)),
            pl.BlockSpec((1, KB), lambda j: (0, j)),
        ],
        out_shape=[
            jax.ShapeDtypeStruct((K, D), jnp.float32),
            jax.ShapeDtypeStruct((1, K), jnp.float32),
        ],
        interpret=interpret,
    )(keys)
    kd, ki = pl.pallas_call(
        _knn_body,
        grid=(NQB, NKB + 1),
        in_specs=[
            pl.BlockSpec((QB, D), lambda i, jj: (i, 0)),
            pl.BlockSpec((QB, 1), lambda i, jj: (i, 0)),
            pl.BlockSpec((KB, D), lambda i, jj: (jnp.minimum(jj, NKB - 1), 0)),
            pl.BlockSpec((1, KB), lambda i, jj: (0, jnp.minimum(jj, NKB - 1))),
        ],
        out_specs=[
            pl.BlockSpec((QB, OUT_W), lambda i, jj: (i, 0)),
            pl.BlockSpec((QB, OUT_W), lambda i, jj: (i, 0)),
        ],
        out_shape=[
            jax.ShapeDtypeStruct((Q, OUT_W), jnp.float32),
            jax.ShapeDtypeStruct((Q, OUT_W), jnp.int32),
        ],
        scratch_shapes=[
            pltpu.VMEM((QB, KB), jnp.float32),
            pltpu.VMEM((QB, KB), jnp.float32),
            pltpu.VMEM((NKB, QB, CW), jnp.float32),
            pltpu.VMEM((NKB, QB, CW), jnp.float32),
        ],
        interpret=interpret,
    )(qs, q2, mn, m2)
    return kd, ki


def kernel(queries, keys):
    kd, ki = _knn(queries, keys)
    knn_dists = kd[:, :NN]
    knn_idx = ki[:, :NN]
    patch_scores = knn_dists[:, 0]
    image_patch_scores = patch_scores.reshape(4, 784)
    image_scores = jnp.max(image_patch_scores, axis=1)
    anomaly_maps = image_patch_scores.reshape(4, 28, 28)
    return (knn_dists, knn_idx, patch_scores, image_scores, anomaly_maps)


# speedup vs baseline: 1.1677x; 1.1677x over previous
import functools

import jax
import jax.numpy as jnp
from jax.experimental import pallas as pl
import jax.experimental.pallas.tpu as pltpu

Q = 3136
K = 8192
D = 1536
QB = 448
NQB = Q // QB
KB = 2048
NKB = K // KB
NN = 9
CW = 32
OUT_W = 16


def _prep_q_body(x_ref, xs_ref, x2_ref):
    x = x_ref[...]
    n = jnp.sqrt(jnp.sum(x * x, axis=1, keepdims=True))
    xn = x / jnp.maximum(n, 1e-12)
    x2_ref[...] = jnp.sum(xn * xn, axis=1, keepdims=True)
    xs_ref[...] = -2.0 * xn


def _prep_k_body(x_ref, xn_ref, x2_ref):
    x = x_ref[...]
    n = jnp.sqrt(jnp.sum(x * x, axis=1, keepdims=True))
    xn = x / jnp.maximum(n, 1e-12)
    xn_ref[...] = xn
    x2_ref[...] = jnp.sum(xn * xn, axis=1)[None, :]


def _knn_body(
    qs_ref, q2_ref, mn_ref, m2_ref, kd_ref, ki_ref, d2a_ref, d2b_ref, cv_ref, ci_ref
):
    jj = pl.program_id(1)

    def compute_into(dst_ref):
        dot2 = jax.lax.dot_general(
            qs_ref[...], mn_ref[...], (((1,), (1,)), ((), ())),
            preferred_element_type=jnp.float32,
        )
        dst_ref[...] = jnp.maximum((q2_ref[...] + m2_ref[...]) + dot2, 0.0)

    def extract_from(src_ref):
        j = jnp.maximum(jj - 1, 0)
        cur = src_ref[...]
        iota = jax.lax.broadcasted_iota(jnp.int32, (QB, KB), 1).astype(jnp.float32)
        base = (j * KB).astype(jnp.float32)
        vals, idxs = [], []
        for _ in range(NN):
            mv = jnp.min(cur, axis=1, keepdims=True)
            e = jnp.where(cur == mv, iota, jnp.float32(KB))
            ii = jnp.min(e, axis=1, keepdims=True)
            vals.append(mv)
            idxs.append(ii + base)
            cur = jnp.where(iota == ii, jnp.float32(jnp.inf), cur)
        pad = CW - NN
        cv = jnp.concatenate(
            vals + [jnp.full((QB, pad), jnp.inf, jnp.float32)], 1
        )
        ci = jnp.concatenate(idxs + [jnp.zeros((QB, pad), jnp.float32)], 1)
        cv_ref[j] = cv
        ci_ref[j] = ci

    @pl.when(jnp.logical_and(jj < NKB, jj % 2 == 0))
    def _():
        compute_into(d2a_ref)

    @pl.when(jnp.logical_and(jj < NKB, jj % 2 == 1))
    def _():
        compute_into(d2b_ref)

    @pl.when(jnp.logical_and(jj >= 1, (jj - 1) % 2 == 0))
    def _():
        extract_from(d2a_ref)

    @pl.when(jnp.logical_and(jj >= 1, (jj - 1) % 2 == 1))
    def _():
        extract_from(d2b_ref)

    @pl.when(jj == NKB)
    def _():
        v = jnp.concatenate([cv_ref[c] for c in range(NKB)], axis=1)
        gi = jnp.concatenate([ci_ref[c] for c in range(NKB)], axis=1)
        pos_id = jax.lax.broadcasted_iota(jnp.int32, (QB, NKB * CW), 1).astype(
            jnp.float32
        )
        out_v, out_i = [], []
        for _ in range(NN):
            mv = jnp.min(v, axis=1, keepdims=True)
            e = jnp.where(v == mv, pos_id, jnp.float32(NKB * CW))
            pos = jnp.min(e, axis=1, keepdims=True)
            hit = pos_id == pos
            ii = jnp.min(jnp.where(hit, gi, jnp.float32(K)), axis=1, keepdims=True)
            out_v.append(mv)
            out_i.append(ii)
            v = jnp.where(hit, jnp.float32(jnp.inf), v)
        opad = OUT_W - NN
        dv = jnp.concatenate(out_v + [jnp.zeros((QB, opad), jnp.float32)], 1)
        di = jnp.concatenate(out_i + [jnp.zeros((QB, opad), jnp.float32)], 1)
        kd_ref[...] = jnp.sqrt(dv + 1e-12)
        ki_ref[...] = di.astype(jnp.int32)


@functools.partial(jax.jit, static_argnames=("interpret",))
def _knn(queries, keys, interpret=False):
    qs, q2 = pl.pallas_call(
        _prep_q_body,
        grid=(NQB,),
        in_specs=[pl.BlockSpec((QB, D), lambda i: (i, 0))],
        out_specs=[
            pl.BlockSpec((QB, D), lambda i: (i, 0)),
            pl.BlockSpec((QB, 1), lambda i: (i, 0)),
        ],
        out_shape=[
            jax.ShapeDtypeStruct((Q, D), jnp.float32),
            jax.ShapeDtypeStruct((Q, 1), jnp.float32),
        ],
        interpret=interpret,
    )(queries)
    mn, m2 = pl.pallas_call(
        _prep_k_body,
        grid=(NKB,),
        in_specs=[pl.BlockSpec((KB, D), lambda j: (j, 0))],
        out_specs=[
            pl.BlockSpec((KB, D), lambda j: (j, 0)),
            pl.BlockSpec((1, KB), lambda j: (0, j)),
        ],
        out_shape=[
            jax.ShapeDtypeStruct((K, D), jnp.float32),
            jax.ShapeDtypeStruct((1, K), jnp.float32),
        ],
        interpret=interpret,
    )(keys)
    kd, ki = pl.pallas_call(
        _knn_body,
        grid=(NQB, NKB + 1),
        in_specs=[
            pl.BlockSpec((QB, D), lambda i, jj: (i, 0)),
            pl.BlockSpec((QB, 1), lambda i, jj: (i, 0)),
            pl.BlockSpec((KB, D), lambda i, jj: (jnp.minimum(jj, NKB - 1), 0)),
            pl.BlockSpec((1, KB), lambda i, jj: (0, jnp.minimum(jj, NKB - 1))),
        ],
        out_specs=[
            pl.BlockSpec((QB, OUT_W), lambda i, jj: (i, 0)),
            pl.BlockSpec((QB, OUT_W), lambda i, jj: (i, 0)),
        ],
        out_shape=[
            jax.ShapeDtypeStruct((Q, OUT_W), jnp.float32),
            jax.ShapeDtypeStruct((Q, OUT_W), jnp.int32),
        ],
        scratch_shapes=[
            pltpu.VMEM((QB, KB), jnp.float32),
            pltpu.VMEM((QB, KB), jnp.float32),
            pltpu.VMEM((NKB, QB, CW), jnp.float32),
            pltpu.VMEM((NKB, QB, CW), jnp.float32),
        ],
        interpret=interpret,
    )(qs, q2, mn, m2)
    return kd, ki


def kernel(queries, keys):
    kd, ki = _knn(queries, keys)
    knn_dists = kd[:, :NN]
    knn_idx = ki[:, :NN]
    patch_scores = knn_dists[:, 0]
    image_patch_scores = patch_scores.reshape(4, 784)
    image_scores = jnp.max(image_patch_scores, axis=1)
    anomaly_maps = image_patch_scores.reshape(4, 28, 28)
    return (knn_dists, knn_idx, patch_scores, image_scores, anomaly_maps)


# speedup vs baseline: 1.1970x; 1.0251x over previous
import functools

import jax
import jax.numpy as jnp
from jax.experimental import pallas as pl
import jax.experimental.pallas.tpu as pltpu

Q = 3136
K = 8192
D = 1536
QB = 448
NQB = Q // QB
KB = 2048
NKB = K // KB
NN = 9
CW = 32
OUT_W = 16


def _prep_q_body(x_ref, xs_ref, x2_ref):
    x = x_ref[...]
    n = jnp.sqrt(jnp.sum(x * x, axis=1, keepdims=True))
    xn = x / jnp.maximum(n, 1e-12)
    x2_ref[...] = jnp.sum(xn * xn, axis=1, keepdims=True)
    xs_ref[...] = -2.0 * xn


def _prep_k_body(x_ref, xn_ref, x2_ref):
    x = x_ref[...]
    n = jnp.sqrt(jnp.sum(x * x, axis=1, keepdims=True))
    xn = x / jnp.maximum(n, 1e-12)
    xn_ref[...] = xn
    x2_ref[...] = jnp.sum(xn * xn, axis=1)[None, :]


def _knn_body(qs_ref, q2_ref, mn_ref, m2_ref, kd_ref, ki_ref, cv_ref, ci_ref):
    j = pl.program_id(1)

    dot2 = jax.lax.dot_general(
        qs_ref[...], mn_ref[...], (((1,), (1,)), ((), ())),
        preferred_element_type=jnp.float32,
    )
    d2 = jnp.maximum((q2_ref[...] + m2_ref[...]) + dot2, 0.0)

    iota = jax.lax.broadcasted_iota(jnp.int32, (QB, KB), 1).astype(jnp.float32)
    base = (j * KB).astype(jnp.float32)
    vals, idxs = [], []
    cur = d2
    for _ in range(NN):
        mv = jnp.min(cur, axis=1, keepdims=True)
        e = jnp.where(cur == mv, iota, jnp.float32(KB))
        ii = jnp.min(e, axis=1, keepdims=True)
        vals.append(mv)
        idxs.append(ii + base)
        cur = jnp.where(iota == ii, jnp.float32(jnp.inf), cur)
    pad = CW - NN
    cv = jnp.concatenate(vals + [jnp.full((QB, pad), jnp.inf, jnp.float32)], 1)
    ci = jnp.concatenate(idxs + [jnp.zeros((QB, pad), jnp.float32)], 1)
    cv_ref[j] = cv
    ci_ref[j] = ci

    @pl.when(j == NKB - 1)
    def _():
        v = jnp.concatenate([cv_ref[c] for c in range(NKB)], axis=1)
        gi = jnp.concatenate([ci_ref[c] for c in range(NKB)], axis=1)
        pos_id = jax.lax.broadcasted_iota(jnp.int32, (QB, NKB * CW), 1).astype(
            jnp.float32
        )
        out_v, out_i = [], []
        for _ in range(NN):
            mv = jnp.min(v, axis=1, keepdims=True)
            e = jnp.where(v == mv, pos_id, jnp.float32(NKB * CW))
            pos = jnp.min(e, axis=1, keepdims=True)
            hit = pos_id == pos
            ii = jnp.min(jnp.where(hit, gi, jnp.float32(K)), axis=1, keepdims=True)
            out_v.append(mv)
            out_i.append(ii)
            v = jnp.where(hit, jnp.float32(jnp.inf), v)
        opad = OUT_W - NN
        dv = jnp.concatenate(out_v + [jnp.zeros((QB, opad), jnp.float32)], 1)
        di = jnp.concatenate(out_i + [jnp.zeros((QB, opad), jnp.float32)], 1)
        kd_ref[...] = jnp.sqrt(dv + 1e-12)
        ki_ref[...] = di.astype(jnp.int32)


@functools.partial(jax.jit, static_argnames=("interpret",))
def _knn(queries, keys, interpret=False):
    qs, q2 = pl.pallas_call(
        _prep_q_body,
        grid=(NQB,),
        in_specs=[pl.BlockSpec((QB, D), lambda i: (i, 0))],
        out_specs=[
            pl.BlockSpec((QB, D), lambda i: (i, 0)),
            pl.BlockSpec((QB, 1), lambda i: (i, 0)),
        ],
        out_shape=[
            jax.ShapeDtypeStruct((Q, D), jnp.float32),
            jax.ShapeDtypeStruct((Q, 1), jnp.float32),
        ],
        interpret=interpret,
    )(queries)
    mn, m2 = pl.pallas_call(
        _prep_k_body,
        grid=(NKB,),
        in_specs=[pl.BlockSpec((KB, D), lambda j: (j, 0))],
        out_specs=[
            pl.BlockSpec((KB, D), lambda j: (j, 0)),
            pl.BlockSpec((1, KB), lambda j: (0, j)),
        ],
        out_shape=[
            jax.ShapeDtypeStruct((K, D), jnp.float32),
            jax.ShapeDtypeStruct((1, K), jnp.float32),
        ],
        interpret=interpret,
    )(keys)
    kd, ki = pl.pallas_call(
        _knn_body,
        grid=(NQB, NKB),
        in_specs=[
            pl.BlockSpec((QB, D), lambda i, j: (i, 0)),
            pl.BlockSpec((QB, 1), lambda i, j: (i, 0)),
            pl.BlockSpec((KB, D), lambda i, j: (j, 0)),
            pl.BlockSpec((1, KB), lambda i, j: (0, j)),
        ],
        out_specs=[
            pl.BlockSpec((QB, OUT_W), lambda i, j: (i, 0)),
            pl.BlockSpec((QB, OUT_W), lambda i, j: (i, 0)),
        ],
        out_shape=[
            jax.ShapeDtypeStruct((Q, OUT_W), jnp.float32),
            jax.ShapeDtypeStruct((Q, OUT_W), jnp.int32),
        ],
        scratch_shapes=[
            pltpu.VMEM((NKB, QB, CW), jnp.float32),
            pltpu.VMEM((NKB, QB, CW), jnp.float32),
        ],
        interpret=interpret,
    )(qs, q2, mn, m2)
    return kd, ki


def kernel(queries, keys):
    kd, ki = _knn(queries, keys)
    knn_dists = kd[:, :NN]
    knn_idx = ki[:, :NN]
    patch_scores = knn_dists[:, 0]
    image_patch_scores = patch_scores.reshape(4, 784)
    image_scores = jnp.max(image_patch_scores, axis=1)
    anomaly_maps = image_patch_scores.reshape(4, 28, 28)
    return (knn_dists, knn_idx, patch_scores, image_scores, anomaly_maps)


# speedup vs baseline: 1.2360x; 1.0326x over previous
import functools

import jax
import jax.numpy as jnp
from jax.experimental import pallas as pl
import jax.experimental.pallas.tpu as pltpu

Q = 3136
K = 8192
D = 1536
QB = 448
NQB = Q // QB
KB = 2048
NKB = K // KB
NN = 9
CW = 32
OUT_W = 16


def _prep_k_body(x_ref, xn_ref, x2_ref):
    x = x_ref[...]
    n = jnp.sqrt(jnp.sum(x * x, axis=1, keepdims=True))
    xn = x / jnp.maximum(n, 1e-12)
    xn_ref[...] = xn
    x2_ref[...] = jnp.sum(xn * xn, axis=1)[None, :]


def _knn_body(q_ref, mn_ref, m2_ref, kd_ref, ki_ref, qs_ref, q2_ref, cv_ref, ci_ref):
    j = pl.program_id(1)

    @pl.when(j == 0)
    def _():
        q = q_ref[...]
        n = jnp.sqrt(jnp.sum(q * q, axis=1, keepdims=True))
        qn = q / jnp.maximum(n, 1e-12)
        q2_ref[...] = jnp.sum(qn * qn, axis=1, keepdims=True)
        qs_ref[...] = -2.0 * qn

    dot2 = jax.lax.dot_general(
        qs_ref[...], mn_ref[...], (((1,), (1,)), ((), ())),
        preferred_element_type=jnp.float32,
    )
    d2 = jnp.maximum((q2_ref[...] + m2_ref[...]) + dot2, 0.0)

    iota = jax.lax.broadcasted_iota(jnp.int32, (QB, KB), 1).astype(jnp.float32)
    base = (j * KB).astype(jnp.float32)
    vals, idxs = [], []
    cur = d2
    for _ in range(NN):
        mv = jnp.min(cur, axis=1, keepdims=True)
        e = jnp.where(cur == mv, iota, jnp.float32(KB))
        ii = jnp.min(e, axis=1, keepdims=True)
        vals.append(mv)
        idxs.append(ii + base)
        cur = jnp.where(iota == ii, jnp.float32(jnp.inf), cur)
    pad = CW - NN
    cv = jnp.concatenate(vals + [jnp.full((QB, pad), jnp.inf, jnp.float32)], 1)
    ci = jnp.concatenate(idxs + [jnp.zeros((QB, pad), jnp.float32)], 1)
    cv_ref[j] = cv
    ci_ref[j] = ci

    @pl.when(j == NKB - 1)
    def _():
        v = jnp.concatenate([cv_ref[c] for c in range(NKB)], axis=1)
        gi = jnp.concatenate([ci_ref[c] for c in range(NKB)], axis=1)
        pos_id = jax.lax.broadcasted_iota(jnp.int32, (QB, NKB * CW), 1).astype(
            jnp.float32
        )
        out_v, out_i = [], []
        for _ in range(NN):
            mv = jnp.min(v, axis=1, keepdims=True)
            e = jnp.where(v == mv, pos_id, jnp.float32(NKB * CW))
            pos = jnp.min(e, axis=1, keepdims=True)
            hit = pos_id == pos
            ii = jnp.min(jnp.where(hit, gi, jnp.float32(K)), axis=1, keepdims=True)
            out_v.append(mv)
            out_i.append(ii)
            v = jnp.where(hit, jnp.float32(jnp.inf), v)
        opad = OUT_W - NN
        dv = jnp.concatenate(out_v + [jnp.zeros((QB, opad), jnp.float32)], 1)
        di = jnp.concatenate(out_i + [jnp.zeros((QB, opad), jnp.float32)], 1)
        kd_ref[...] = jnp.sqrt(dv + 1e-12)
        ki_ref[...] = di.astype(jnp.int32)


@functools.partial(jax.jit, static_argnames=("interpret",))
def _knn(queries, keys, interpret=False):
    mn, m2 = pl.pallas_call(
        _prep_k_body,
        grid=(NKB,),
        in_specs=[pl.BlockSpec((KB, D), lambda j: (j, 0))],
        out_specs=[
            pl.BlockSpec((KB, D), lambda j: (j, 0)),
            pl.BlockSpec((1, KB), lambda j: (0, j)),
        ],
        out_shape=[
            jax.ShapeDtypeStruct((K, D), jnp.float32),
            jax.ShapeDtypeStruct((1, K), jnp.float32),
        ],
        interpret=interpret,
    )(keys)
    kd, ki = pl.pallas_call(
        _knn_body,
        grid=(NQB, NKB),
        in_specs=[
            pl.BlockSpec((QB, D), lambda i, j: (i, 0)),
            pl.BlockSpec((KB, D), lambda i, j: (j, 0)),
            pl.BlockSpec((1, KB), lambda i, j: (0, j)),
        ],
        out_specs=[
            pl.BlockSpec((QB, OUT_W), lambda i, j: (i, 0)),
            pl.BlockSpec((QB, OUT_W), lambda i, j: (i, 0)),
        ],
        out_shape=[
            jax.ShapeDtypeStruct((Q, OUT_W), jnp.float32),
            jax.ShapeDtypeStruct((Q, OUT_W), jnp.int32),
        ],
        scratch_shapes=[
            pltpu.VMEM((QB, D), jnp.float32),
            pltpu.VMEM((QB, 1), jnp.float32),
            pltpu.VMEM((NKB, QB, CW), jnp.float32),
            pltpu.VMEM((NKB, QB, CW), jnp.float32),
        ],
        interpret=interpret,
    )(queries, mn, m2)
    return kd, ki


def kernel(queries, keys):
    kd, ki = _knn(queries, keys)
    knn_dists = kd[:, :NN]
    knn_idx = ki[:, :NN]
    patch_scores = knn_dists[:, 0]
    image_patch_scores = patch_scores.reshape(4, 784)
    image_scores = jnp.max(image_patch_scores, axis=1)
    anomaly_maps = image_patch_scores.reshape(4, 28, 28)
    return (knn_dists, knn_idx, patch_scores, image_scores, anomaly_maps)
